# baseline (speedup 1.0000x reference)
import functools

import jax
import jax.numpy as jnp
from jax.experimental import pallas as pl
from jax.experimental.pallas import tpu as pltpu

_COMMITMENT_COST = 0.25


def _vq_body(x_ref, q_ref, loss_ref, perp_ref, *, steps, scale):
    i = pl.program_id(0)
    x = x_ref[...]
    q_ref[...] = jnp.zeros_like(x)

    @pl.when(i == 0)
    def _init():
        loss_ref[0, 0] = 0.0
        perp_ref[0, 0] = 1.0

    xr = x.reshape(x.shape[0] // 16, 16, x.shape[1])
    partial = jnp.sum(xr * xr, axis=0)
    loss_ref[0, 0] += jnp.sum(partial)

    @pl.when(i == steps - 1)
    def _finish():
        loss_ref[0, 0] = loss_ref[0, 0] * scale


def kernel(inputs, weight):
    b, t, d = inputs.shape
    n = b * t
    flat = inputs.reshape(n, d)
    chunk = 8192
    steps = n // chunk
    scale = (1.0 + _COMMITMENT_COST) / float(n * d)
    quantized, loss, perplexity = pl.pallas_call(
        functools.partial(_vq_body, steps=steps, scale=scale),
        grid=(steps,),
        in_specs=[pl.BlockSpec((chunk, d), lambda i: (i, 0))],
        out_specs=(
            pl.BlockSpec((chunk, d), lambda i: (i, 0)),
            pl.BlockSpec(memory_space=pltpu.SMEM),
            pl.BlockSpec(memory_space=pltpu.SMEM),
        ),
        out_shape=(
            jax.ShapeDtypeStruct((n, d), inputs.dtype),
            jax.ShapeDtypeStruct((1, 1), jnp.float32),
            jax.ShapeDtypeStruct((1, 1), jnp.float32),
        ),
    )(flat)
    return quantized.reshape(inputs.shape), loss[0, 0], perplexity[0, 0]
